# baseline (speedup 1.0000x reference)
import functools

import jax
import jax.numpy as jnp
from jax import lax
from jax.experimental import pallas as pl
from jax.experimental.pallas import tpu as pltpu
from jax.experimental.pallas import tpu_sc as plsc

_N_NODES = 10000
_N_EDGES = 320000
_P = 128
_D = 16
_N_OUT = 2048

_NC = 2
_NS = 16
_NW = _NC * _NS

_CHUNK = 128
_EP = 2560 * _CHUNK
_ROWS = _EP // _CHUNK
_RPW = _ROWS // _NW
_HRP = _ROWS // _NS
_NPAD = 10240
_SEG = _NPAD // _NS
_DUMMY = _N_NODES
_SCRAP = _N_OUT
_ACC = _N_OUT + _CHUNK
_ASEG = _ACC // _NS
_OPT = _N_OUT // _NS
_KC = 512
_CPW = (_NPAD + 2 * _KC - 1) // _KC

_NBUF = 4
_AHEAD = 2


def _sc_mega_body(dst_hbm, src_hbm, h_hbm, acc_out, deg_out,
                  hidx_v, didx_v, sidx_v, cd_v, cs_v, hist_v,
                  mtmp_v, dis_v, hrow_v, rows_v, gsem, ssem,
                  hist_sh, hp_sh, acc_sh):
    cid = lax.axis_index("c")
    sid = lax.axis_index("s")
    wid = cid * _NS + sid
    one16f = jnp.full((16,), 1.0, jnp.float32)

    def zh(i, c):
        hist_v[pl.ds(i * 16, 16)] = jnp.zeros((16,), jnp.float32)
        return c

    lax.fori_loop(0, _NPAD // 16, zh, 0)

    def za(i, c):
        rows_v[0, i, :] = jnp.zeros((16,), jnp.float32)
        return c

    lax.fori_loop(0, _CHUNK, za, 0)
    pltpu.sync_copy(rows_v.at[0].at[pl.ds(0, _CHUNK)],
                    acc_sh.at[pl.ds(sid * _ASEG, _CHUNK)])
    pltpu.sync_copy(rows_v.at[0].at[pl.ds(0, _ASEG - _CHUNK)],
                    acc_sh.at[pl.ds(sid * _ASEG + _CHUNK, _ASEG - _CHUNK)])

    for half in range(2):
        pltpu.sync_copy(dst_hbm.at[pl.ds(sid * _HRP + half * _RPW, _RPW)],
                        hidx_v)

        def hrow(r, c):
            for v in range(_CHUNK // 16):
                d = hidx_v[r, pl.ds(v * 16, 16)]
                plsc.addupdate_scatter(hist_v, [d], one16f)
            return c

        lax.fori_loop(0, _RPW, hrow, 0)

    pltpu.sync_copy(dst_hbm.at[pl.ds(wid * _RPW, _RPW)], didx_v)
    pltpu.sync_copy(src_hbm.at[pl.ds(wid * _RPW, _RPW)], sidx_v)

    def crow(r, off):
        for v in range(_CHUNK // 16):
            d = didx_v[r, pl.ds(v * 16, 16)]
            s = sidx_v[r, pl.ds(v * 16, 16)]
            m = d < _N_OUT
            mi = jnp.where(m, jnp.full((16,), 1, jnp.int32),
                           jnp.zeros((16,), jnp.int32))
            pos = off + plsc.cumsum(mi) - mi
            pr = pos >> 9
            pc = pos & 511
            plsc.store_scatter(cd_v, [pr, pc], d, mask=m)
            plsc.store_scatter(cs_v, [pr, pc], s, mask=m)
            off = off + plsc.all_reduce_population_count(m)
        return off

    off = lax.fori_loop(0, _RPW, crow, jnp.zeros((16,), jnp.int32))

    dvec = jnp.full((16,), _SCRAP, jnp.int32)
    svec = jnp.zeros((16,), jnp.int32)
    lanes = lax.iota(jnp.int32, 16)
    for k in range(_KC // 16):
        pp = off + (k * 16) + lanes
        plsc.store_scatter(cd_v, [pp >> 9, pp & 511], dvec)
        plsc.store_scatter(cs_v, [pp >> 9, pp & 511], svec)
    nch = jnp.max((off + (_KC - 1)) >> 9)

    pltpu.sync_copy(hist_v, hist_sh.at[sid])
    plsc.subcore_barrier()
    pltpu.sync_copy(hist_sh.at[0].at[pl.ds(sid * _SEG, _SEG)], mtmp_v.at[0])

    def merge(t, c):
        pltpu.sync_copy(hist_sh.at[t].at[pl.ds(sid * _SEG, _SEG)],
                        mtmp_v.at[1])

        def madd(i, c2):
            mtmp_v[0, pl.ds(i * 16, 16)] = (mtmp_v[0, pl.ds(i * 16, 16)]
                                            + mtmp_v[1, pl.ds(i * 16, 16)])
            return c2

        lax.fori_loop(0, _SEG // 16, madd, 0)
        return c

    lax.fori_loop(1, _NS, merge, 0)

    def dblk(g, c):
        x = mtmp_v[0, pl.ds(g * 16, 16)] + 1.0
        i0 = plsc.bitcast(x, jnp.int32)
        y = plsc.bitcast(jnp.full((16,), 0x5F3759DF, jnp.int32) - (i0 >> 1),
                         jnp.float32)
        hx = x * (-0.5)
        for _ in range(3):
            y = y * (1.5 + hx * y * y)
        dis_v[pl.ds(g * 16, 16)] = y
        return c

    lax.fori_loop(0, _SEG // 16, dblk, 0)

    pltpu.sync_copy(h_hbm.at[pl.ds(sid * _SEG, _SEG)], hrow_v)

    def sblk(g, gvec):
        for i in range(16):
            r = g * 16 + i
            sp = plsc.load_gather(dis_v, [gvec * 16 + i])
            hrow_v[r, :] = hrow_v[r, :] * sp
        return gvec + 1

    lax.fori_loop(0, _SEG // 16, sblk, jnp.zeros((16,), jnp.int32))
    pltpu.sync_copy(hrow_v, hp_sh.at[pl.ds(sid * _SEG, _SEG)])

    @pl.when(cid == 0)
    def _():
        pltpu.sync_copy(mtmp_v.at[0], deg_out.at[sid])

    plsc.subcore_barrier()

    for j in range(_AHEAD):
        @pl.when(j < nch)
        def _(j=j):
            pltpu.async_copy(hp_sh.at[cs_v.at[j]],
                             rows_v.at[j % _NBUF], gsem.at[j % _NBUF])

    def outer(g, c):
        for b in range(_NBUF):
            j = g * _NBUF + b

            @pl.when(j < nch)
            def _():
                pltpu.make_async_copy(hp_sh.at[cs_v.at[j]],
                                      rows_v.at[b], gsem.at[b]).wait()
                pltpu.async_copy(rows_v.at[b], acc_sh.at[cd_v.at[j]],
                                 ssem.at[b], add=True)

            jn = j + _AHEAD
            bn = (b + _AHEAD) % _NBUF

            @pl.when(jnp.logical_and(jn >= _NBUF, jn < nch))
            def _():
                pltpu.make_async_copy(rows_v.at[bn],
                                      acc_sh.at[cd_v.at[jn - _NBUF]],
                                      ssem.at[bn]).wait()

            @pl.when(jn < nch)
            def _():
                pltpu.async_copy(hp_sh.at[cs_v.at[jn]], rows_v.at[bn],
                                 gsem.at[bn])
        return c

    lax.fori_loop(0, (nch + _NBUF - 1) >> 2, outer, 0)
    for b in range(_NBUF):
        @pl.when(b < nch)
        def _(b=b):
            pltpu.make_async_copy(rows_v.at[b], acc_sh.at[cd_v.at[0]],
                                  ssem.at[b]).wait()

    plsc.subcore_barrier()
    pltpu.sync_copy(acc_sh.at[pl.ds(sid * _OPT, _OPT)],
                    acc_out.at[cid].at[pl.ds(sid * _OPT, _OPT)])


_sc_mega = functools.partial(
    pl.kernel,
    out_type=[
        jax.ShapeDtypeStruct((_NC, _N_OUT, _D), jnp.float32),
        jax.ShapeDtypeStruct((_NS, _SEG), jnp.float32),
    ],
    compiler_params=pltpu.CompilerParams(use_tc_tiling_on_sc=False,
                                         needs_layout_passes=False),
    mesh=plsc.VectorSubcoreMesh(core_axis_name="c", subcore_axis_name="s"),
    scratch_types=[
        pltpu.VMEM((_RPW, _CHUNK), jnp.int32),
        pltpu.VMEM((_RPW, _CHUNK), jnp.int32),
        pltpu.VMEM((_RPW, _CHUNK), jnp.int32),
        pltpu.VMEM((_CPW, _KC), jnp.int32),
        pltpu.VMEM((_CPW, _KC), jnp.int32),
        pltpu.VMEM((_NPAD,), jnp.float32),
        pltpu.VMEM((2, _SEG), jnp.float32),
        pltpu.VMEM((_SEG,), jnp.float32),
        pltpu.VMEM((_SEG, _D), jnp.float32),
        pltpu.VMEM((_NBUF, _KC, _D), jnp.float32),
        pltpu.SemaphoreType.DMA((_NBUF,)),
        pltpu.SemaphoreType.DMA((_NBUF,)),
        pltpu.VMEM_SHARED((_NS, _NPAD), jnp.float32),
        pltpu.VMEM_SHARED((_NPAD, _D), jnp.float32),
        pltpu.VMEM_SHARED((_ACC, _D), jnp.float32),
    ],
)(_sc_mega_body)


_TC_BLK = 1024


def _tc_matmul_body(x_ref, w_ref, h_ref):
    h_ref[...] = jnp.dot(x_ref[...], w_ref[...],
                         preferred_element_type=jnp.float32)


def _tc_matmul(x, w):
    return pl.pallas_call(
        _tc_matmul_body,
        grid=(_NPAD // _TC_BLK,),
        in_specs=[
            pl.BlockSpec((_TC_BLK, _P), lambda i: (i, 0)),
            pl.BlockSpec((_P, _D), lambda i: (0, 0)),
        ],
        out_specs=pl.BlockSpec((_TC_BLK, _D), lambda i: (i, 0)),
        out_shape=jax.ShapeDtypeStruct((_NPAD, _D), jnp.float32),
    )(x, w)


def _tc_epilogue_body(a0_ref, a1_ref, dg_ref, h_ref, b_ref, o_ref):
    dis = lax.rsqrt(dg_ref[...] + 1.0)
    o_ref[...] = (dis * (a0_ref[...] + a1_ref[...])
                  + (dis * dis) * h_ref[...] + b_ref[...])


def _tc_epilogue(a0, a1, dg, h, b2):
    return pl.pallas_call(
        _tc_epilogue_body,
        out_shape=jax.ShapeDtypeStruct((_N_OUT, _D), jnp.float32),
    )(a0, a1, dg, h, b2)


def kernel(x, pd_edge_index, W, b):
    ei = pd_edge_index.astype(jnp.int32)
    src = ei[0]
    dst = ei[1]
    pad = _EP - _N_EDGES
    src_p = jnp.concatenate([src, jnp.zeros((pad,), jnp.int32)]).reshape(_ROWS, _CHUNK)
    dst_p = jnp.concatenate([dst, jnp.full((pad,), _DUMMY, jnp.int32)]).reshape(_ROWS, _CHUNK)

    h = _tc_matmul(x, W)
    acc2, deg = _sc_mega(dst_p, src_p, h)
    dg = deg.reshape(_NPAD)[:_N_OUT].reshape(_N_OUT, 1)
    out = _tc_epilogue(acc2[0], acc2[1], dg, h[:_N_OUT], b.reshape(1, _D))
    return out


# speedup vs baseline: 1.1313x; 1.1313x over previous
import functools

import jax
import jax.numpy as jnp
from jax import lax
from jax.experimental import pallas as pl
from jax.experimental.pallas import tpu as pltpu
from jax.experimental.pallas import tpu_sc as plsc

_N_NODES = 10000
_N_EDGES = 320000
_P = 128
_D = 16
_N_OUT = 2048

_NC = 2
_NS = 16
_NW = _NC * _NS

_CHUNK = 128
_EP = 2560 * _CHUNK
_ROWS = _EP // _CHUNK
_RPW = _ROWS // _NW
_DEG_PAD = 10240
_SEG = _DEG_PAD // _NS
_DUMMY = _N_NODES
_SCRAP = _N_OUT
_ACC = _N_OUT + _CHUNK
_ASEG = _ACC // _NS
_OPT = _N_OUT // _NS
_CPW = _RPW + 1

_NBUF = 8
_AHEAD = 6
_BINFLY = 8


def _sc_filter_body(dst_hbm, src_hbm, deg_out, csrc_out, cdst_out, cnt_out,
                    didx_v, sidx_v, cd_v, cs_v, one_v, zro_v, cnt_v,
                    deg_sh, bsem):
    cid = lax.axis_index("c")
    sid = lax.axis_index("s")
    wid = cid * _NS + sid

    def z(i, c):
        zro_v[pl.ds(i * 16, 16)] = jnp.zeros((16,), jnp.float32)
        return c

    lax.fori_loop(0, _SEG // 16, z, 0)

    def o(i, c):
        one_v[pl.ds(i * 16, 16)] = jnp.full((16,), 1.0, jnp.float32)
        return c

    lax.fori_loop(0, _CHUNK // 16, o, 0)

    pltpu.sync_copy(zro_v, deg_sh.at[pl.ds(sid * _SEG, _SEG)])
    pltpu.sync_copy(dst_hbm.at[pl.ds(wid * _RPW, _RPW)], didx_v)
    pltpu.sync_copy(src_hbm.at[pl.ds(wid * _RPW, _RPW)], sidx_v)
    plsc.subcore_barrier()

    def row(r, off):
        pltpu.async_copy(one_v, deg_sh.at[didx_v.at[r]], bsem, add=True)

        @pl.when(r >= _BINFLY)
        def _():
            pltpu.make_async_copy(one_v, deg_sh.at[didx_v.at[0]], bsem).wait()

        for v in range(_CHUNK // 16):
            d = didx_v[r, pl.ds(v * 16, 16)]
            s = sidx_v[r, pl.ds(v * 16, 16)]
            m = d < _N_OUT
            mi = jnp.where(m, jnp.full((16,), 1, jnp.int32),
                           jnp.zeros((16,), jnp.int32))
            pos = off + plsc.cumsum(mi) - mi
            pr = pos >> 7
            pc = pos & 127
            plsc.store_scatter(cd_v, [pr, pc], d, mask=m)
            plsc.store_scatter(cs_v, [pr, pc], s, mask=m)
            off = off + plsc.all_reduce_population_count(m)
        return off

    off = lax.fori_loop(0, _RPW, row,
                        jnp.zeros((16,), jnp.int32))
    for _ in range(_BINFLY):
        pltpu.make_async_copy(one_v, deg_sh.at[didx_v.at[0]], bsem).wait()

    dvec = jnp.full((16,), _SCRAP, jnp.int32)
    svec = jnp.zeros((16,), jnp.int32)
    lanes = lax.iota(jnp.int32, 16)
    for k in range(_CHUNK // 16):
        pp = off + (k * 16) + lanes
        plsc.store_scatter(cd_v, [pp >> 7, pp & 127], dvec)
        plsc.store_scatter(cs_v, [pp >> 7, pp & 127], svec)
    cnt_v[...] = (off + (_CHUNK - 1)) >> 7

    pltpu.sync_copy(cd_v.at[pl.ds(0, _RPW)], cdst_out.at[wid])
    pltpu.sync_copy(cs_v.at[pl.ds(0, _RPW)], csrc_out.at[wid])
    pltpu.sync_copy(cnt_v, cnt_out.at[wid])
    plsc.subcore_barrier()
    pltpu.sync_copy(deg_sh.at[pl.ds(sid * _SEG, _SEG)],
                    deg_out.at[cid].at[pl.ds(sid * _SEG, _SEG)])


_sc_filter = functools.partial(
    pl.kernel,
    out_type=[
        jax.ShapeDtypeStruct((_NC, _DEG_PAD), jnp.float32),
        jax.ShapeDtypeStruct((_NW, _RPW, _CHUNK), jnp.int32),
        jax.ShapeDtypeStruct((_NW, _RPW, _CHUNK), jnp.int32),
        jax.ShapeDtypeStruct((_NW, 16), jnp.int32),
    ],
    compiler_params=pltpu.CompilerParams(use_tc_tiling_on_sc=False,
                                         needs_layout_passes=False),
    mesh=plsc.VectorSubcoreMesh(core_axis_name="c", subcore_axis_name="s"),
    scratch_types=[
        pltpu.VMEM((_RPW, _CHUNK), jnp.int32),
        pltpu.VMEM((_RPW, _CHUNK), jnp.int32),
        pltpu.VMEM((_CPW, _CHUNK), jnp.int32),
        pltpu.VMEM((_CPW, _CHUNK), jnp.int32),
        pltpu.VMEM((_CHUNK,), jnp.float32),
        pltpu.VMEM((_SEG,), jnp.float32),
        pltpu.VMEM((16,), jnp.int32),
        pltpu.VMEM_SHARED((_DEG_PAD,), jnp.float32),
        pltpu.SemaphoreType.DMA,
    ],
)(_sc_filter_body)


def _sc_aggregate_body(hp_hbm, csrc_hbm, cdst_hbm, cnt_hbm, acc_out,
                       sidx_v, didx_v, rows_v, cnt_v, acc_sh, gsem, ssem):
    cid = lax.axis_index("c")
    sid = lax.axis_index("s")
    wid = cid * _NS + sid

    def z(i, c):
        rows_v[0, i, :] = jnp.zeros((16,), jnp.float32)
        return c

    lax.fori_loop(0, _CHUNK, z, 0)
    pltpu.sync_copy(rows_v.at[0], acc_sh.at[pl.ds(sid * _ASEG, _CHUNK)])
    pltpu.sync_copy(rows_v.at[0].at[pl.ds(0, _ASEG - _CHUNK)],
                    acc_sh.at[pl.ds(sid * _ASEG + _CHUNK, _ASEG - _CHUNK)])

    pltpu.sync_copy(csrc_hbm.at[wid], sidx_v)
    pltpu.sync_copy(cdst_hbm.at[wid], didx_v)
    pltpu.sync_copy(cnt_hbm.at[wid], cnt_v)
    nch = jnp.max(cnt_v[...])
    plsc.subcore_barrier()

    for j in range(_AHEAD):
        @pl.when(j < nch)
        def _(j=j):
            pltpu.async_copy(hp_hbm.at[sidx_v.at[j]], rows_v.at[j],
                             gsem.at[j])

    def outer(g, c):
        for b in range(_NBUF):
            j = g * _NBUF + b

            @pl.when(j < nch)
            def _():
                pltpu.make_async_copy(hp_hbm.at[sidx_v.at[j]], rows_v.at[b],
                                      gsem.at[b]).wait()
                pltpu.async_copy(rows_v.at[b], acc_sh.at[didx_v.at[j]],
                                 ssem.at[b], add=True)

            jn = j + _AHEAD
            bn = (b + _AHEAD) % _NBUF

            @pl.when(jnp.logical_and(jn >= _NBUF, jn < nch))
            def _():
                pltpu.make_async_copy(rows_v.at[bn],
                                      acc_sh.at[didx_v.at[jn - _NBUF]],
                                      ssem.at[bn]).wait()

            @pl.when(jn < nch)
            def _():
                pltpu.async_copy(hp_hbm.at[sidx_v.at[jn]], rows_v.at[bn],
                                 gsem.at[bn])
        return c

    lax.fori_loop(0, (nch + _NBUF - 1) >> 3, outer, 0)
    for b in range(_NBUF):
        @pl.when(b < nch)
        def _(b=b):
            pltpu.make_async_copy(rows_v.at[b], acc_sh.at[didx_v.at[0]],
                                  ssem.at[b]).wait()

    plsc.subcore_barrier()
    pltpu.sync_copy(acc_sh.at[pl.ds(sid * _OPT, _OPT)],
                    acc_out.at[cid].at[pl.ds(sid * _OPT, _OPT)])


_sc_aggregate = functools.partial(
    pl.kernel,
    out_type=jax.ShapeDtypeStruct((_NC, _N_OUT, _D), jnp.float32),
    compiler_params=pltpu.CompilerParams(use_tc_tiling_on_sc=False,
                                         needs_layout_passes=False),
    mesh=plsc.VectorSubcoreMesh(core_axis_name="c", subcore_axis_name="s"),
    scratch_types=[
        pltpu.VMEM((_RPW, _CHUNK), jnp.int32),
        pltpu.VMEM((_RPW, _CHUNK), jnp.int32),
        pltpu.VMEM((_NBUF, _CHUNK, _D), jnp.float32),
        pltpu.VMEM((16,), jnp.int32),
        pltpu.VMEM_SHARED((_ACC, _D), jnp.float32),
        pltpu.SemaphoreType.DMA((_NBUF,)),
        pltpu.SemaphoreType.DMA((_NBUF,)),
    ],
)(_sc_aggregate_body)


_TC_BLK = 1000


def _tc_transform_body(x_ref, w_ref, d0_ref, d1_ref, hp_ref, dis_ref):
    d = d0_ref[...] + d1_ref[...] + 1.0
    dis = lax.rsqrt(d)
    h = jnp.dot(x_ref[...], w_ref[...], preferred_element_type=jnp.float32)
    hp_ref[...] = h * dis
    dis_ref[...] = dis


def _tc_transform(x, w, deg0, deg1):
    return pl.pallas_call(
        _tc_transform_body,
        grid=(_N_NODES // _TC_BLK,),
        in_specs=[
            pl.BlockSpec((_TC_BLK, _P), lambda i: (i, 0)),
            pl.BlockSpec((_P, _D), lambda i: (0, 0)),
            pl.BlockSpec((_TC_BLK, 1), lambda i: (i, 0)),
            pl.BlockSpec((_TC_BLK, 1), lambda i: (i, 0)),
        ],
        out_specs=[
            pl.BlockSpec((_TC_BLK, _D), lambda i: (i, 0)),
            pl.BlockSpec((_TC_BLK, 1), lambda i: (i, 0)),
        ],
        out_shape=[
            jax.ShapeDtypeStruct((_N_NODES, _D), jnp.float32),
            jax.ShapeDtypeStruct((_N_NODES, 1), jnp.float32),
        ],
    )(x, w, deg0, deg1)


def _tc_epilogue_body(a0_ref, a1_ref, hp_ref, dis_ref, b_ref, o_ref):
    o_ref[...] = (dis_ref[...] * (a0_ref[...] + a1_ref[...] + hp_ref[...])
                  + b_ref[...])


def _tc_epilogue(a0, a1, hp, dis, b2):
    return pl.pallas_call(
        _tc_epilogue_body,
        out_shape=jax.ShapeDtypeStruct((_N_OUT, _D), jnp.float32),
    )(a0, a1, hp, dis, b2)


def kernel(x, pd_edge_index, W, b):
    ei = pd_edge_index.astype(jnp.int32)
    src = ei[0]
    dst = ei[1]
    pad = _EP - _N_EDGES
    src_p = jnp.concatenate([src, jnp.zeros((pad,), jnp.int32)]).reshape(_ROWS, _CHUNK)
    dst_p = jnp.concatenate([dst, jnp.full((pad,), _DUMMY, jnp.int32)]).reshape(_ROWS, _CHUNK)

    deg2, csrc, cdst, cnt = _sc_filter(dst_p, src_p)
    deg0 = deg2[0, :_N_NODES].reshape(_N_NODES, 1)
    deg1 = deg2[1, :_N_NODES].reshape(_N_NODES, 1)
    hp, dis = _tc_transform(x, W, deg0, deg1)
    acc2 = _sc_aggregate(hp, csrc, cdst, cnt)
    out = _tc_epilogue(acc2[0], acc2[1], hp[:_N_OUT], dis[:_N_OUT],
                       b.reshape(1, _D))
    return out
